# baseline (speedup 1.0000x reference)
import jax
import jax.numpy as jnp
from jax.experimental import pallas as pl
from jax.experimental.pallas import tpu as pltpu

SOM_H, SOM_W, D = 64, 64, 512
HW = SOM_H * SOM_W
BATCH = 4096
TB = 512
CHUNK = 512
NC = HW // CHUNK


def _som_kernel(x_ref, wn_ref, coord_ref, idx_ref, wsq_ref):
    @pl.when(pl.program_id(0) == 0)
    def _():
        wn = wn_ref[...]
        wsq_ref[...] = (jnp.sum(wn * wn, axis=1) * 0.25)[None, :]

    x = x_ref[...]
    x_sq = jnp.sum(x * x, axis=1, keepdims=True)
    mins = []
    args = []
    for c in range(NC):
        wc = wn_ref[pl.ds(c * CHUNK, CHUNK), :]
        cross2 = jax.lax.dot_general(
            x, wc, (((1,), (1,)), ((), ())),
            preferred_element_type=jnp.float32,
        )
        dist = (x_sq + cross2) + wsq_ref[:, c * CHUNK:(c + 1) * CHUNK]
        mins.append(jnp.min(dist, axis=1))
        args.append(jnp.argmin(dist, axis=1).astype(jnp.int32) + c * CHUNK)
    m = jnp.stack(mins, axis=1)
    a = jnp.stack(args, axis=1)
    jj = jnp.argmin(m, axis=1)
    sel = jax.lax.broadcasted_iota(jnp.int32, (TB, NC), 1) == jj[:, None]
    idx = jnp.sum(jnp.where(sel, a, 0), axis=1).astype(jnp.int32)
    idx_ref[...] = idx[:, None]
    coord_ref[...] = jnp.stack([idx // SOM_W, idx % SOM_W], axis=1)


def kernel(x, weights):
    wneg = (-2.0 * weights).reshape(HW, D)
    grid = (BATCH // TB,)
    coords, idx = pl.pallas_call(
        _som_kernel,
        grid=grid,
        in_specs=[
            pl.BlockSpec((TB, D), lambda i: (i, 0)),
            pl.BlockSpec((HW, D), lambda i: (0, 0)),
        ],
        out_specs=[
            pl.BlockSpec((TB, 2), lambda i: (i, 0)),
            pl.BlockSpec((TB, 1), lambda i: (i, 0)),
        ],
        out_shape=[
            jax.ShapeDtypeStruct((BATCH, 2), jnp.int32),
            jax.ShapeDtypeStruct((BATCH, 1), jnp.int32),
        ],
        scratch_shapes=[pltpu.VMEM((1, HW), jnp.float32)],
    )(x, wneg)
    return coords, idx[:, 0]


# speedup vs baseline: 1.7357x; 1.7357x over previous
import jax
import jax.numpy as jnp
from jax.experimental import pallas as pl
from jax.experimental.pallas import tpu as pltpu

SOM_H, SOM_W, D = 64, 64, 512
HW = SOM_H * SOM_W
BATCH = 4096
TB = 512


def _som_kernel(x_ref, wn_ref, coord_ref, idx_ref, wsq_ref):
    @pl.when(pl.program_id(0) == 0)
    def _():
        wn = wn_ref[...]
        wsq_ref[...] = (jnp.sum(wn * wn, axis=1) * 0.25)[None, :]

    x = x_ref[...]
    x_sq = jnp.sum(x * x, axis=1, keepdims=True)
    cross2 = jax.lax.dot_general(
        x, wn_ref[...], (((1,), (1,)), ((), ())),
        preferred_element_type=jnp.float32,
    )
    dist = (x_sq + cross2) + wsq_ref[...]
    idx = jnp.argmin(dist, axis=1).astype(jnp.int32)
    idx_ref[...] = idx[:, None]
    coord_ref[...] = jnp.stack([idx // SOM_W, idx % SOM_W], axis=1)


def kernel(x, weights):
    wneg = (-2.0 * weights).reshape(HW, D)
    grid = (BATCH // TB,)
    coords, idx = pl.pallas_call(
        _som_kernel,
        grid=grid,
        in_specs=[
            pl.BlockSpec((TB, D), lambda i: (i, 0)),
            pl.BlockSpec((HW, D), lambda i: (0, 0)),
        ],
        out_specs=[
            pl.BlockSpec((TB, 2), lambda i: (i, 0)),
            pl.BlockSpec((TB, 1), lambda i: (i, 0)),
        ],
        out_shape=[
            jax.ShapeDtypeStruct((BATCH, 2), jnp.int32),
            jax.ShapeDtypeStruct((BATCH, 1), jnp.int32),
        ],
        scratch_shapes=[pltpu.VMEM((1, HW), jnp.float32)],
    )(x, wneg)
    return coords, idx[:, 0]


# speedup vs baseline: 1.9585x; 1.1283x over previous
import jax
import jax.numpy as jnp
from jax.experimental import pallas as pl
from jax.experimental.pallas import tpu as pltpu

SOM_H, SOM_W, D = 64, 64, 512
HW = SOM_H * SOM_W
BATCH = 4096
TB = 512


def _som_kernel(x_ref, w_ref, coord_ref, idx_ref, wsq_ref):
    @pl.when(pl.program_id(0) == 0)
    def _():
        w = w_ref[...]
        wsq_ref[...] = jnp.sum(w * w, axis=1)[None, :]

    x = x_ref[...]
    x_sq = jnp.sum(x * x, axis=1, keepdims=True)
    cross = jax.lax.dot_general(
        x, w_ref[...], (((1,), (1,)), ((), ())),
        preferred_element_type=jnp.float32,
    )
    dist = (x_sq - 2.0 * cross) + wsq_ref[...]
    idx = jnp.argmin(dist, axis=1).astype(jnp.int32)
    idx_ref[...] = idx[:, None]
    coord_ref[...] = jnp.stack([idx // SOM_W, idx % SOM_W], axis=1)


def kernel(x, weights):
    wneg = weights.reshape(HW, D)
    grid = (BATCH // TB,)
    coords, idx = pl.pallas_call(
        _som_kernel,
        grid=grid,
        in_specs=[
            pl.BlockSpec((TB, D), lambda i: (i, 0)),
            pl.BlockSpec((HW, D), lambda i: (0, 0)),
        ],
        out_specs=[
            pl.BlockSpec((TB, 2), lambda i: (i, 0)),
            pl.BlockSpec((TB, 1), lambda i: (i, 0)),
        ],
        out_shape=[
            jax.ShapeDtypeStruct((BATCH, 2), jnp.int32),
            jax.ShapeDtypeStruct((BATCH, 1), jnp.int32),
        ],
        scratch_shapes=[pltpu.VMEM((1, HW), jnp.float32)],
    )(x, wneg)
    return coords, idx[:, 0]


# speedup vs baseline: 2.0928x; 1.0686x over previous
import jax
import jax.numpy as jnp
from jax.experimental import pallas as pl
from jax.experimental.pallas import tpu as pltpu

SOM_H, SOM_W, D = 64, 64, 512
HW = SOM_H * SOM_W
BATCH = 4096
TB = 1024


def _som_kernel(x_ref, w_ref, coord_ref, idx_ref, wsq_ref):
    @pl.when(pl.program_id(0) == 0)
    def _():
        w = w_ref[...]
        wsq_ref[...] = jnp.sum(w * w, axis=1)[None, :]

    x = x_ref[...]
    x_sq = jnp.sum(x * x, axis=1, keepdims=True)
    cross = jax.lax.dot_general(
        x, w_ref[...], (((1,), (1,)), ((), ())),
        preferred_element_type=jnp.float32,
    )
    dist = (x_sq - 2.0 * cross) + wsq_ref[...]
    idx = jnp.argmin(dist, axis=1).astype(jnp.int32)
    idx_ref[...] = idx[:, None]
    coord_ref[...] = jnp.stack([idx // SOM_W, idx % SOM_W], axis=1)


def kernel(x, weights):
    wneg = weights.reshape(HW, D)
    grid = (BATCH // TB,)
    coords, idx = pl.pallas_call(
        _som_kernel,
        grid=grid,
        in_specs=[
            pl.BlockSpec((TB, D), lambda i: (i, 0)),
            pl.BlockSpec((HW, D), lambda i: (0, 0)),
        ],
        out_specs=[
            pl.BlockSpec((TB, 2), lambda i: (i, 0)),
            pl.BlockSpec((TB, 1), lambda i: (i, 0)),
        ],
        out_shape=[
            jax.ShapeDtypeStruct((BATCH, 2), jnp.int32),
            jax.ShapeDtypeStruct((BATCH, 1), jnp.int32),
        ],
        scratch_shapes=[pltpu.VMEM((1, HW), jnp.float32)],
    )(x, wneg)
    return coords, idx[:, 0]
